# Initial kernel scaffold; baseline (speedup 1.0000x reference)
import jax
import jax.numpy as jnp
from jax.experimental import pallas as pl


def kernel(x, indices):
    raise NotImplementedError("write your pallas kernel here")


# speedup vs baseline: 3.0602x; 3.0602x over previous
import dataclasses
import functools

import jax
import jax.numpy as jnp
from jax import lax
from jax.experimental import pallas as pl
from jax.experimental.pallas import tpu as pltpu
from jax.experimental.pallas import tpu_sc as plsc

_B, _C, _N = 8, 64, 32768
_NC, _NS = 2, 16
_NW = _NC * _NS
_WPB = _NW // _B
_CPW = _C // _WPB
_ROWS = _B * _C


def _compiler_params():
    cp = pltpu.CompilerParams()
    if "needs_layout_passes" in pltpu.CompilerParams.__dataclass_fields__:
        cp = dataclasses.replace(cp, needs_layout_passes=False)
    return cp


@jax.jit
def _sc_permute(x2d, indices):
    mesh = plsc.VectorSubcoreMesh(core_axis_name="c", subcore_axis_name="s")

    @functools.partial(
        pl.kernel,
        compiler_params=_compiler_params(),
        out_type=jax.ShapeDtypeStruct((_ROWS, _N), jnp.float32),
        mesh=mesh,
        scratch_types=[
            pltpu.VMEM((_N,), jnp.int32),
            pltpu.VMEM((_N,), jnp.float32),
            pltpu.VMEM((_N,), jnp.float32),
        ],
    )
    def k(x_hbm, idx_hbm, out_hbm, idx_v, x_v, out_v):
        wid = lax.axis_index("s") * _NC + lax.axis_index("c")
        b = wid // _WPB
        cg = wid % _WPB
        pltpu.sync_copy(idx_hbm.at[b], idx_v)
        for ci in range(_CPW):
            row = b * _C + cg * _CPW + ci
            pltpu.sync_copy(x_hbm.at[row], x_v)

            @pl.loop(0, _N, step=16)
            def _(j):
                vidx = idx_v[pl.ds(j, 16)]
                vx = x_v[pl.ds(j, 16)]
                plsc.store_scatter(out_v, [vidx], vx)

            pltpu.sync_copy(out_v, out_hbm.at[row])

    return k(x2d, indices)


def kernel(x, indices):
    out = _sc_permute(x.reshape(_ROWS, _N), indices)
    return out.reshape(_B, _C, _N)


# speedup vs baseline: 7.3879x; 2.4142x over previous
import dataclasses
import functools

import jax
import jax.numpy as jnp
from jax import lax
from jax.experimental import pallas as pl
from jax.experimental.pallas import tpu as pltpu
from jax.experimental.pallas import tpu_sc as plsc

_B, _C, _N = 8, 64, 32768
_NC, _NS = 2, 16
_NW = _NC * _NS
_WPB = _NW // _B
_CPW = _C // _WPB
_ROWS = _B * _C


def _compiler_params():
    cp = pltpu.CompilerParams()
    if "needs_layout_passes" in pltpu.CompilerParams.__dataclass_fields__:
        cp = dataclasses.replace(cp, needs_layout_passes=False)
    return cp


@jax.jit
def _sc_permute(x2d, indices):
    mesh = plsc.VectorSubcoreMesh(core_axis_name="c", subcore_axis_name="s")

    @functools.partial(
        pl.kernel,
        compiler_params=_compiler_params(),
        out_type=jax.ShapeDtypeStruct((_ROWS, _N), jnp.float32),
        mesh=mesh,
        scratch_types=[
            pltpu.VMEM((_N,), jnp.int32),
            pltpu.VMEM((_N,), jnp.float32),
            pltpu.VMEM((_N,), jnp.float32),
        ],
    )
    def k(x_hbm, idx_hbm, out_hbm, idx_v, x_v, out_v):
        wid = lax.axis_index("s") * _NC + lax.axis_index("c")
        b = wid // _WPB
        cg = wid % _WPB
        pltpu.sync_copy(idx_hbm.at[b], idx_v)
        for ci in range(_CPW):
            row = b * _C + cg * _CPW + ci
            pltpu.sync_copy(x_hbm.at[row], x_v)

            @plsc.parallel_loop(0, _N, 16, unroll=8)
            def _(j):
                vidx = idx_v[pl.ds(j, 16)]
                vx = x_v[pl.ds(j, 16)]
                plsc.store_scatter(out_v, [vidx], vx)

            pltpu.sync_copy(out_v, out_hbm.at[row])

    return k(x2d, indices)


def kernel(x, indices):
    out = _sc_permute(x.reshape(_ROWS, _N), indices)
    return out.reshape(_B, _C, _N)
